# baseline (speedup 1.0000x reference)
import jax
import jax.numpy as jnp
from jax.experimental import pallas as pl

_TOKENS = 32768
_DIM = 768
_EXPERTS = 8
_BLOCK = 4096


def _gate_body(x_ref, w_ref, b_ref, idx_ref, val_ref):
    xb = x_ref[...]
    logits = jnp.dot(xb, w_ref[...], preferred_element_type=jnp.float32)
    logits = logits + b_ref[...]
    lt = logits.T
    ids = jax.lax.broadcasted_iota(jnp.int32, lt.shape, 0)
    m1 = jnp.max(lt, axis=0, keepdims=True)
    z = jnp.sum(jnp.exp(lt - m1), axis=0, keepdims=True)
    i1 = jnp.min(jnp.where(lt == m1, ids, _EXPERTS), axis=0, keepdims=True)
    masked = jnp.where(ids == i1, -jnp.inf, lt)
    m2 = jnp.max(masked, axis=0, keepdims=True)
    i2 = jnp.min(jnp.where(masked == m2, ids, _EXPERTS), axis=0, keepdims=True)
    inv_z = 1.0 / z
    v2 = jnp.exp(m2 - m1) * inv_z
    idx_ref[...] = jnp.concatenate([i1, i2], axis=0).T
    val_ref[...] = jnp.concatenate([inv_z, v2], axis=0).T


def kernel(x, W_gate, b_gate):
    n_blocks = _TOKENS // _BLOCK
    b2d = b_gate.reshape(1, _EXPERTS)
    grid_spec = pl.GridSpec(
        grid=(n_blocks,),
        in_specs=[
            pl.BlockSpec((_BLOCK, _DIM), lambda i: (i, 0)),
            pl.BlockSpec((_DIM, _EXPERTS), lambda i: (0, 0)),
            pl.BlockSpec((1, _EXPERTS), lambda i: (0, 0)),
        ],
        out_specs=[
            pl.BlockSpec((_BLOCK, 2), lambda i: (i, 0)),
            pl.BlockSpec((_BLOCK, 2), lambda i: (i, 0)),
        ],
    )
    idx, val = pl.pallas_call(
        _gate_body,
        grid_spec=grid_spec,
        out_shape=[
            jax.ShapeDtypeStruct((_TOKENS, 2), jnp.int32),
            jax.ShapeDtypeStruct((_TOKENS, 2), jnp.float32),
        ],
    )(x, W_gate, b2d)
    return idx, val


# speedup vs baseline: 1.9548x; 1.9548x over previous
import jax
import jax.numpy as jnp
from jax.experimental import pallas as pl

_TOKENS = 32768
_DIM = 768
_EXPERTS = 8
_BLOCK = 4096


def _gate_body(x_ref, w_ref, b_ref, idx_ref, val_ref):
    xb = x_ref[...]
    logits = jnp.dot(xb, w_ref[...], preferred_element_type=jnp.float32)
    logits = logits + b_ref[...]
    lt = logits.T
    ids = jax.lax.broadcasted_iota(jnp.int32, lt.shape, 0)
    m1 = jnp.max(lt, axis=0, keepdims=True)
    z = jnp.sum(jnp.exp(lt - m1), axis=0, keepdims=True)
    i1 = jnp.min(jnp.where(lt == m1, ids, _EXPERTS), axis=0, keepdims=True)
    masked = jnp.where(ids == i1, -jnp.inf, lt)
    m2 = jnp.max(masked, axis=0, keepdims=True)
    i2 = jnp.min(jnp.where(masked == m2, ids, _EXPERTS), axis=0, keepdims=True)
    inv_z = 1.0 / z
    v2 = jnp.exp(m2 - m1) * inv_z
    idx_ref[...] = jnp.concatenate([i1, i2], axis=0)
    val_ref[...] = jnp.concatenate([inv_z, v2], axis=0)


def kernel(x, W_gate, b_gate):
    n_blocks = _TOKENS // _BLOCK
    b2d = b_gate.reshape(1, _EXPERTS)
    grid_spec = pl.GridSpec(
        grid=(n_blocks,),
        in_specs=[
            pl.BlockSpec((_BLOCK, _DIM), lambda i: (i, 0)),
            pl.BlockSpec((_DIM, _EXPERTS), lambda i: (0, 0)),
            pl.BlockSpec((1, _EXPERTS), lambda i: (0, 0)),
        ],
        out_specs=[
            pl.BlockSpec((2, _BLOCK), lambda i: (0, i)),
            pl.BlockSpec((2, _BLOCK), lambda i: (0, i)),
        ],
    )
    idx_t, val_t = pl.pallas_call(
        _gate_body,
        grid_spec=grid_spec,
        out_shape=[
            jax.ShapeDtypeStruct((2, _TOKENS), jnp.int32),
            jax.ShapeDtypeStruct((2, _TOKENS), jnp.float32),
        ],
    )(x, W_gate, b2d)
    return idx_t.T, val_t.T
